# Initial kernel scaffold; baseline (speedup 1.0000x reference)
import jax
import jax.numpy as jnp
from jax.experimental import pallas as pl


def kernel(data, news_title_indexes, news_entity_indexes):
    raise NotImplementedError("write your pallas kernel here")


# speedup vs baseline: 3.0235x; 3.0235x over previous
import functools

import jax
import jax.numpy as jnp
from jax import lax
from jax.experimental import pallas as pl
from jax.experimental.pallas import tpu as pltpu
from jax.experimental.pallas import tpu_sc as plsc

B = 4096
NEWS_NUM = 5
TITLE_LEN = 30
ENTITY_LEN = 5
UC, US, UN = 8, 4, 5

TITLE_PAD = 32
ENT_PAD = 16

NC, NS = 2, 16
NW = NC * NS
CHUNK = 128

N_TGT = B * NEWS_NUM
N_USR = B * UC * US * UN
TGT_CHUNKS = N_TGT // (NW * CHUNK)
USR_CHUNKS = N_USR // (NW * CHUNK)
TGT_PER_W = N_TGT // NW
USR_PER_W = N_USR // NW

_mesh = plsc.VectorSubcoreMesh(core_axis_name="c", subcore_axis_name="s")


@functools.partial(
    pl.kernel,
    out_type=[
        jax.ShapeDtypeStruct((N_TGT, TITLE_PAD), jnp.int32),
        jax.ShapeDtypeStruct((N_TGT, ENT_PAD), jnp.int32),
        jax.ShapeDtypeStruct((N_USR, TITLE_PAD), jnp.int32),
        jax.ShapeDtypeStruct((N_USR, ENT_PAD), jnp.int32),
    ],
    mesh=_mesh,
    compiler_params=pltpu.CompilerParams(use_tc_tiling_on_sc=False),
    scratch_types=[
        pltpu.VMEM((TGT_CHUNKS, CHUNK), jnp.int32),
        pltpu.VMEM((USR_CHUNKS, CHUNK), jnp.int32),
        pltpu.VMEM((CHUNK, TITLE_PAD), jnp.int32),
        pltpu.VMEM((CHUNK, ENT_PAD), jnp.int32),
        pltpu.SemaphoreType.DMA,
    ],
)
def _gather_kernel(tgt_idx_hbm, usr_idx_hbm, title_hbm, ent_hbm,
                   tgt_title_out, tgt_ent_out, usr_title_out, usr_ent_out,
                   tgt_idx_v, usr_idx_v, title_buf, ent_buf, sem):
    wid = lax.axis_index("s") * NC + lax.axis_index("c")
    pltpu.sync_copy(tgt_idx_hbm.at[wid], tgt_idx_v)
    pltpu.sync_copy(usr_idx_hbm.at[wid], usr_idx_v)

    tgt_base = wid * TGT_PER_W
    usr_base = wid * USR_PER_W

    def make_body(idx_v, title_out, ent_out, base):
        def body(j, carry):
            idx = idx_v.at[j]
            out_off = base + j * CHUNK
            pltpu.async_copy(title_hbm.at[idx], title_buf, sem).wait()
            pltpu.sync_copy(title_buf, title_out.at[pl.ds(out_off, CHUNK)])
            pltpu.async_copy(ent_hbm.at[idx], ent_buf, sem).wait()
            pltpu.sync_copy(ent_buf, ent_out.at[pl.ds(out_off, CHUNK)])
            return carry
        return body

    lax.fori_loop(0, TGT_CHUNKS,
                  make_body(tgt_idx_v, tgt_title_out, tgt_ent_out, tgt_base), 0)
    lax.fori_loop(0, USR_CHUNKS,
                  make_body(usr_idx_v, usr_title_out, usr_ent_out, usr_base), 0)


def kernel(data, news_title_indexes, news_entity_indexes):
    tgt_idx = data[:, :NEWS_NUM].reshape(NW, TGT_CHUNKS, CHUNK)
    usr_idx = data[:, NEWS_NUM * 5 + UC + UC * US:].reshape(NW, USR_CHUNKS, CHUNK)

    title_pad = jnp.pad(news_title_indexes, ((0, 0), (0, TITLE_PAD - TITLE_LEN)))
    ent_pad = jnp.pad(news_entity_indexes, ((0, 0), (0, ENT_PAD - ENTITY_LEN)))

    tgt_title, tgt_ent, usr_title, usr_ent = _gather_kernel(
        tgt_idx, usr_idx, title_pad, ent_pad)

    return (
        tgt_title[:, :TITLE_LEN].reshape(B, NEWS_NUM, TITLE_LEN),
        tgt_ent[:, :ENTITY_LEN].reshape(B, NEWS_NUM, ENTITY_LEN),
        usr_title[:, :TITLE_LEN].reshape(B, UC, US, UN, TITLE_LEN),
        usr_ent[:, :ENTITY_LEN].reshape(B, UC, US, UN, ENTITY_LEN),
    )


# speedup vs baseline: 6.2514x; 2.0676x over previous
import functools

import jax
import jax.numpy as jnp
import numpy as np
from jax import lax
from jax.experimental import pallas as pl
from jax.experimental.pallas import tpu as pltpu
from jax.experimental.pallas import tpu_sc as plsc

B = 4096
NEWS_NUM = 5
TITLE_LEN = 30
ENTITY_LEN = 5
UC, US, UN = 8, 4, 5

COMB_W = 48

NC, NS = 2, 16
NW = NC * NS
CHUNK = 128

N_TGT = B * NEWS_NUM
N_USR = B * UC * US * UN
TGT_CHUNKS = N_TGT // (NW * CHUNK)
USR_CHUNKS = N_USR // (NW * CHUNK)
TGT_PER_W = N_TGT // NW
USR_PER_W = N_USR // NW

T_WORDS = CHUNK * TITLE_LEN
E_WORDS = CHUNK * ENTITY_LEN
L = 16

_t_j = np.arange(T_WORDS, dtype=np.int32)
_T_ROW = _t_j // TITLE_LEN
_T_COL = _t_j % TITLE_LEN
_e_j = np.arange(E_WORDS, dtype=np.int32)
_E_ROW = _e_j // ENTITY_LEN
_E_COL = TITLE_LEN + _e_j % ENTITY_LEN

_mesh = plsc.VectorSubcoreMesh(core_axis_name="c", subcore_axis_name="s")


@functools.partial(
    pl.kernel,
    out_type=[
        jax.ShapeDtypeStruct((N_TGT * TITLE_LEN,), jnp.int32),
        jax.ShapeDtypeStruct((N_TGT * ENTITY_LEN,), jnp.int32),
        jax.ShapeDtypeStruct((N_USR * TITLE_LEN,), jnp.int32),
        jax.ShapeDtypeStruct((N_USR * ENTITY_LEN,), jnp.int32),
    ],
    mesh=_mesh,
    compiler_params=pltpu.CompilerParams(use_tc_tiling_on_sc=False,
                                         needs_layout_passes=False),
    scratch_types=[
        pltpu.VMEM((TGT_CHUNKS, CHUNK), jnp.int32),
        pltpu.VMEM((USR_CHUNKS, CHUNK), jnp.int32),
        pltpu.VMEM((CHUNK, COMB_W), jnp.int32),
        pltpu.VMEM((CHUNK, COMB_W), jnp.int32),
        pltpu.VMEM((T_WORDS,), jnp.int32),
        pltpu.VMEM((T_WORDS,), jnp.int32),
        pltpu.VMEM((E_WORDS,), jnp.int32),
        pltpu.VMEM((E_WORDS,), jnp.int32),
        pltpu.VMEM((T_WORDS,), jnp.int32),
        pltpu.VMEM((T_WORDS,), jnp.int32),
        pltpu.VMEM((E_WORDS,), jnp.int32),
        pltpu.VMEM((E_WORDS,), jnp.int32),
        pltpu.SemaphoreType.DMA,
        pltpu.SemaphoreType.DMA,
        pltpu.SemaphoreType.DMA,
        pltpu.SemaphoreType.DMA,
    ],
)
def _gather_kernel(tgt_idx_hbm, usr_idx_hbm, comb_hbm,
                   trow_hbm, tcol_hbm, erow_hbm, ecol_hbm,
                   tgt_title_out, tgt_ent_out, usr_title_out, usr_ent_out,
                   tgt_idx_v, usr_idx_v, gbuf_a, gbuf_b,
                   ct_a, ct_b, ce_a, ce_b,
                   trow_v, tcol_v, erow_v, ecol_v,
                   gsem_a, gsem_b, ssem_a, ssem_b):
    wid = lax.axis_index("s") * NC + lax.axis_index("c")
    pltpu.sync_copy(tgt_idx_hbm.at[wid], tgt_idx_v)
    pltpu.sync_copy(usr_idx_hbm.at[wid], usr_idx_v)
    pltpu.sync_copy(trow_hbm, trow_v)
    pltpu.sync_copy(tcol_hbm, tcol_v)
    pltpu.sync_copy(erow_hbm, erow_v)
    pltpu.sync_copy(ecol_hbm, ecol_v)

    def compact(gbuf, ct, ce):
        def tbody(k, carry):
            for u in range(8):
                off = k * 128 + u * 16
                rows = trow_v[pl.ds(off, L)]
                cols = tcol_v[pl.ds(off, L)]
                ct[pl.ds(off, L)] = plsc.load_gather(gbuf, [rows, cols])
            return carry
        lax.fori_loop(0, T_WORDS // 128, tbody, 0)

        def ebody(k, carry):
            for u in range(8):
                off = k * 128 + u * 16
                rows = erow_v[pl.ds(off, L)]
                cols = ecol_v[pl.ds(off, L)]
                ce[pl.ds(off, L)] = plsc.load_gather(gbuf, [rows, cols])
            return carry
        lax.fori_loop(0, E_WORDS // 128, ebody, 0)

    tgt_t_base = wid * TGT_PER_W * TITLE_LEN
    tgt_e_base = wid * TGT_PER_W * ENTITY_LEN

    def tgt_body(j, carry):
        pltpu.async_copy(comb_hbm.at[tgt_idx_v.at[j]], gbuf_a, gsem_a).wait()
        compact(gbuf_a, ct_a, ce_a)
        pltpu.sync_copy(ct_a, tgt_title_out.at[pl.ds(tgt_t_base + j * T_WORDS, T_WORDS)])
        pltpu.sync_copy(ce_a, tgt_ent_out.at[pl.ds(tgt_e_base + j * E_WORDS, E_WORDS)])
        return carry
    lax.fori_loop(0, TGT_CHUNKS, tgt_body, 0)

    usr_t_base = wid * USR_PER_W * TITLE_LEN
    usr_e_base = wid * USR_PER_W * ENTITY_LEN

    def start_gather(c, gbuf, gsem):
        cc = jnp.minimum(c, USR_CHUNKS - 1)
        pltpu.async_copy(comb_hbm.at[usr_idx_v.at[cc]], gbuf, gsem)

    def wait_gather(gbuf, gsem):
        pltpu.make_async_copy(comb_hbm.at[pl.ds(0, CHUNK)], gbuf, gsem).wait()

    def t_slice(c):
        return usr_title_out.at[pl.ds(usr_t_base + c * T_WORDS, T_WORDS)]

    def e_slice(c):
        return usr_ent_out.at[pl.ds(usr_e_base + c * E_WORDS, E_WORDS)]

    def drain_store_pair(ct, ce, ssem):
        pltpu.make_async_copy(ct, t_slice(0), ssem).wait()
        pltpu.make_async_copy(ce, e_slice(0), ssem).wait()

    start_gather(0, gbuf_a, gsem_a)
    start_gather(1, gbuf_b, gsem_b)
    pltpu.async_copy(ct_a, t_slice(0), ssem_a)
    pltpu.async_copy(ce_a, e_slice(0), ssem_a)
    pltpu.async_copy(ct_b, t_slice(1), ssem_b)
    pltpu.async_copy(ce_b, e_slice(1), ssem_b)

    def usr_body(t, carry):
        for par, gbuf, gsem, ct, ce, ssem in (
            (0, gbuf_a, gsem_a, ct_a, ce_a, ssem_a),
            (1, gbuf_b, gsem_b, ct_b, ce_b, ssem_b),
        ):
            c = 2 * t + par
            drain_store_pair(ct, ce, ssem)
            wait_gather(gbuf, gsem)
            compact(gbuf, ct, ce)
            start_gather(c + 2, gbuf, gsem)
            pltpu.async_copy(ct, t_slice(c), ssem)
            pltpu.async_copy(ce, e_slice(c), ssem)
        return carry
    lax.fori_loop(0, USR_CHUNKS // 2, usr_body, 0)

    wait_gather(gbuf_a, gsem_a)
    wait_gather(gbuf_b, gsem_b)
    drain_store_pair(ct_a, ce_a, ssem_a)
    drain_store_pair(ct_b, ce_b, ssem_b)


def kernel(data, news_title_indexes, news_entity_indexes):
    tgt_idx = data[:, :NEWS_NUM].reshape(NW, TGT_CHUNKS, CHUNK)
    usr_idx = data[:, NEWS_NUM * 5 + UC + UC * US:].reshape(NW, USR_CHUNKS, CHUNK)

    comb = jnp.concatenate(
        [news_title_indexes, news_entity_indexes,
         jnp.zeros((news_title_indexes.shape[0], COMB_W - TITLE_LEN - ENTITY_LEN),
                   jnp.int32)], axis=1)

    tgt_title, tgt_ent, usr_title, usr_ent = _gather_kernel(
        tgt_idx, usr_idx, comb,
        jnp.asarray(_T_ROW), jnp.asarray(_T_COL),
        jnp.asarray(_E_ROW), jnp.asarray(_E_COL))

    return (
        tgt_title.reshape(B, NEWS_NUM, TITLE_LEN),
        tgt_ent.reshape(B, NEWS_NUM, ENTITY_LEN),
        usr_title.reshape(B, UC, US, UN, TITLE_LEN),
        usr_ent.reshape(B, UC, US, UN, ENTITY_LEN),
    )
